# baseline (speedup 1.0000x reference)
import functools

import jax
import jax.numpy as jnp
from jax import lax
from jax.experimental import pallas as pl
from jax.experimental.pallas import tpu as pltpu
from jax.experimental.pallas import tpu_sc as plsc

_NUM_CORES = 2
_NUM_SUBCORES = 16
_NUM_WORKERS = _NUM_CORES * _NUM_SUBCORES
_LANES = 16

_GROUP = 128
_GROUPS_PER_CHUNK = 4
_CHUNK = _GROUP * _GROUPS_PER_CHUNK


def _make_lookup(num_rows: int, vocab: int, dim: int):
  assert num_rows % (_NUM_WORKERS * _GROUP) == 0
  groups_per_w = num_rows // (_NUM_WORKERS * _GROUP)
  assert groups_per_w % _GROUPS_PER_CHUNK == 0
  nchunk = groups_per_w // _GROUPS_PER_CHUNK
  assert nchunk % 2 == 0
  rows_per_w = groups_per_w * _GROUP

  mesh = plsc.VectorSubcoreMesh(
      core_axis_name="c", subcore_axis_name="s", num_cores=_NUM_CORES,
      num_subcores=_NUM_SUBCORES)

  @functools.partial(
      pl.kernel,
      mesh=mesh,
      out_type=jax.ShapeDtypeStruct((num_rows, dim), jnp.float32),
      compiler_params=pltpu.CompilerParams(
          use_tc_tiling_on_sc=False, skip_device_barrier=True),
      scratch_types=[
          pltpu.VMEM((groups_per_w, _GROUP), jnp.int32),
          pltpu.VMEM((_CHUNK, dim), jnp.float32),
          pltpu.VMEM((_CHUNK, dim), jnp.float32),
          pltpu.SemaphoreType.DMA,
          pltpu.SemaphoreType.DMA,
          pltpu.SemaphoreType.DMA,
          pltpu.SemaphoreType.DMA,
      ],
  )
  def lookup(table_hbm, idx_hbm, out_hbm, idx_v, rows0, rows1,
             gsem0, gsem1, osem0, osem1):
    wid = lax.axis_index("s") * _NUM_CORES + lax.axis_index("c")
    base_grp = wid * groups_per_w
    base_row = wid * rows_per_w

    rows = (rows0, rows1)
    gsem = (gsem0, gsem1)
    osem = (osem0, osem1)

    pltpu.sync_copy(idx_hbm.at[pl.ds(base_grp, groups_per_w)], idx_v)

    def fire(chunk, b):
      for g in range(_GROUPS_PER_CHUNK):
        pltpu.async_copy(
            table_hbm.at[idx_v.at[chunk * _GROUPS_PER_CHUNK + g]],
            rows[b].at[pl.ds(g * _GROUP, _GROUP)],
            gsem[b])

    def drain_gather(b):
      pltpu.make_async_copy(
          table_hbm.at[pl.ds(0, _CHUNK)], rows[b], gsem[b]).wait()

    def wait_out(b):
      pltpu.make_async_copy(
          rows[b], out_hbm.at[pl.ds(0, _CHUNK)], osem[b]).wait()

    fire(0, 0)

    @pl.loop(0, nchunk, step=2)
    def _chunks(c):
      for b in range(2):
        cc = c + b

        @pl.when(cc >= 1)
        def _():
          wait_out(1 - b)

        @pl.when(cc + 1 < nchunk)
        def _():
          fire(cc + 1, 1 - b)

        drain_gather(b)

        @pl.loop(0, _CHUNK, unroll=8)
        def _scale(r):
          for j in range(dim // _LANES):
            sl = pl.ds(j * _LANES, _LANES)
            rows[b][r, sl] = rows[b][r, sl] * 8.0

        pltpu.async_copy(
            rows[b], out_hbm.at[pl.ds(base_row + cc * _CHUNK, _CHUNK)],
            osem[b])

    wait_out((nchunk - 1) % 2)

  return lookup


def kernel(x, input_embedding_table):
  batch, seq = x.shape
  vocab, dim = input_embedding_table.shape
  num_rows = batch * seq
  idx2d = x.reshape(num_rows // _GROUP, _GROUP).astype(jnp.int32)
  lookup = _make_lookup(num_rows, vocab, dim)
  out = lookup(input_embedding_table, idx2d)
  return out.reshape(batch, seq, dim)


# speedup vs baseline: 1.2240x; 1.2240x over previous
import functools

import jax
import jax.numpy as jnp
from jax import lax
from jax.experimental import pallas as pl
from jax.experimental.pallas import tpu as pltpu
from jax.experimental.pallas import tpu_sc as plsc

_NUM_CORES = 2
_NUM_SUBCORES = 16
_NUM_WORKERS = _NUM_CORES * _NUM_SUBCORES
_LANES = 16

_GROUP = 128
_GROUPS_PER_CHUNK = 2
_CHUNK = _GROUP * _GROUPS_PER_CHUNK
_WIDE = 128


def _make_lookup(num_rows: int, vocab: int, dim: int):
  assert num_rows % (_NUM_WORKERS * _GROUP) == 0
  groups_per_w = num_rows // (_NUM_WORKERS * _GROUP)
  assert groups_per_w % _GROUPS_PER_CHUNK == 0
  nchunk = groups_per_w // _GROUPS_PER_CHUNK
  assert nchunk % 2 == 0
  rows_per_w = groups_per_w * _GROUP

  mesh = plsc.VectorSubcoreMesh(
      core_axis_name="c", subcore_axis_name="s", num_cores=_NUM_CORES,
      num_subcores=_NUM_SUBCORES)

  @functools.partial(
      pl.kernel,
      mesh=mesh,
      out_type=jax.ShapeDtypeStruct((num_rows, _WIDE), jnp.float32),
      compiler_params=pltpu.CompilerParams(use_tc_tiling_on_sc=True),
      scratch_types=[
          pltpu.VMEM((groups_per_w, _GROUP), jnp.int32),
          pltpu.VMEM((_CHUNK, _WIDE), jnp.float32),
          pltpu.VMEM((_CHUNK, _WIDE), jnp.float32),
          pltpu.SemaphoreType.DMA,
          pltpu.SemaphoreType.DMA,
          pltpu.SemaphoreType.DMA,
          pltpu.SemaphoreType.DMA,
      ],
  )
  def lookup(table_hbm, idx_hbm, out_hbm, idx_v, rows0, rows1,
             gsem0, gsem1, osem0, osem1):
    wid = lax.axis_index("s") * _NUM_CORES + lax.axis_index("c")
    base_grp = wid * groups_per_w
    base_row = wid * rows_per_w

    rows = (rows0, rows1)
    gsem = (gsem0, gsem1)
    osem = (osem0, osem1)

    pltpu.sync_copy(idx_hbm.at[pl.ds(base_grp, groups_per_w)], idx_v)

    def fire(chunk, b):
      for g in range(_GROUPS_PER_CHUNK):
        pltpu.async_copy(
            table_hbm.at[idx_v.at[chunk * _GROUPS_PER_CHUNK + g]],
            rows[b].at[pl.ds(g * _GROUP, _GROUP)],
            gsem[b])

    def drain_gather(b):
      pltpu.make_async_copy(
          table_hbm.at[pl.ds(0, _CHUNK)], rows[b], gsem[b]).wait()

    def wait_out(b):
      pltpu.make_async_copy(
          rows[b], out_hbm.at[pl.ds(0, _CHUNK)], osem[b]).wait()

    fire(0, 0)

    @pl.loop(0, nchunk, step=2)
    def _chunks(c):
      for b in range(2):
        cc = c + b

        @pl.when(cc >= 1)
        def _():
          wait_out(1 - b)

        @pl.when(cc + 1 < nchunk)
        def _():
          fire(cc + 1, 1 - b)

        drain_gather(b)

        @pl.loop(0, _CHUNK, unroll=8)
        def _scale(r):
          for j in range(dim // _LANES):
            sl = pl.ds(j * _LANES, _LANES)
            rows[b][r, sl] = rows[b][r, sl] * 8.0

        pltpu.async_copy(
            rows[b], out_hbm.at[pl.ds(base_row + cc * _CHUNK, _CHUNK)],
            osem[b])

    wait_out((nchunk - 1) % 2)

  return lookup


def kernel(x, input_embedding_table):
  batch, seq = x.shape
  vocab, dim = input_embedding_table.shape
  num_rows = batch * seq
  idx2d = x.reshape(num_rows // _GROUP, _GROUP).astype(jnp.int32)
  table_wide = jnp.pad(input_embedding_table, ((0, 0), (0, _WIDE - dim)))
  lookup = _make_lookup(num_rows, vocab, dim)
  out = lookup(table_wide, idx2d)
  return out[:, :dim].reshape(batch, seq, dim)
